# baseline (speedup 1.0000x reference)
import jax
import jax.numpy as jnp
from jax import lax
from jax.experimental import pallas as pl
from jax.experimental.pallas import tpu as pltpu

EPS = 1e-6
K = 20
SLOPE = 0.2
HI = lax.Precision.HIGHEST
NEG = -3.4e38


def _mm(a, b):
    return jnp.dot(a, b, precision=HI, preferred_element_type=jnp.float32)


def _mmd(a, b):
    return jnp.dot(a, b, precision=lax.Precision.DEFAULT,
                   preferred_element_type=jnp.float32)


def _full_spec(arr):
    nd = arr.ndim
    return pl.BlockSpec(arr.shape, lambda *a, _nd=nd: (0,) * _nd)


def _kron3(w):
    return jnp.kron(jnp.eye(3, dtype=w.dtype), w)


def _sel(o):
    s = jnp.tile(jnp.eye(o, dtype=jnp.float32), (3, 1))
    return s, s.T


def _knn_body(x_ref, idx_ref):
    xb = x_ref[0]
    n = xb.shape[0]
    g = lax.dot_general(xb, xb, (((1,), (1,)), ((), ())),
                        precision=lax.Precision.DEFAULT,
                        preferred_element_type=jnp.float32)
    xx = jnp.sum(xb * xb, axis=1, keepdims=True)
    xxr = jnp.sum(xb * xb, axis=1).reshape(1, n)
    pd = 2.0 * g - xx - xxr
    iota = lax.broadcasted_iota(jnp.int32, (n, n), 1)
    for j in range(K):
        mx = jnp.max(pd, axis=1, keepdims=True)
        am = jnp.min(jnp.where(pd >= mx, iota, n), axis=1, keepdims=True)
        idx_ref[0, :, pl.ds(j, 1)] = am
        pd = jnp.where(iota == am, NEG, pd)


def _knn(nodes):
    b, n, d = nodes.shape
    return pl.pallas_call(
        _knn_body,
        grid=(b,),
        in_specs=[pl.BlockSpec((1, n, d), lambda i: (i, 0, 0))],
        out_specs=pl.BlockSpec((1, n, K), lambda i: (i, 0, 0)),
        out_shape=jax.ShapeDtypeStruct((b, n, K), jnp.int32),
    )(nodes)


def _gather_body(x_ref, idx_ref, o_ref):
    j = pl.program_id(1)
    xb = x_ref[0]
    n = xb.shape[0]
    ii = idx_ref[0]
    sel = lax.broadcasted_iota(jnp.int32, (n, K), 1) == j
    col = jnp.sum(jnp.where(sel, ii, 0), axis=1, keepdims=True)
    oh = (lax.broadcasted_iota(jnp.int32, (n, n), 1) == col).astype(jnp.float32)
    o_ref[0, 0] = _mm(oh, xb)


def _gather(nodes, idx):
    b, n, d = nodes.shape
    return pl.pallas_call(
        _gather_body,
        grid=(b, K),
        in_specs=[pl.BlockSpec((1, n, d), lambda i, j: (i, 0, 0)),
                  pl.BlockSpec((1, n, K), lambda i, j: (i, 0, 0))],
        out_specs=pl.BlockSpec((1, 1, n, d), lambda i, j: (i, j, 0, 0)),
        out_shape=jax.ShapeDtypeStruct((b, K, n, d), jnp.float32),
    )(nodes, idx)


def _vnl(p, d, ssum, ssq, cnt, s, r):
    m = ssum / cnt
    var = ssq / cnt - m * m
    n = jnp.sqrt(_mm(p * p, s)) + EPS
    nb = (n - m) / jnp.sqrt(var + 1e-5)
    pb = p * _mm(nb / n, r)
    dot = _mm(pb * d, s)
    dns = _mm(d * d, s)
    coef = jnp.where(dot < 0, dot / (dns + EPS), 0.0) * (1.0 - SLOPE)
    return pb - _mm(coef, r) * d


def _acc(ref, row, v):
    s = jnp.sum(v, axis=0, keepdims=True)
    q = jnp.sum(v * v, axis=0, keepdims=True)
    ref[pl.ds(row, 1), :] += s
    ref[pl.ds(row + 1, 1), :] += q


def _edge_block(feats, nodes, ws, s21, r21, hi_a=False):
    mma = _mm if hi_a else _mmd
    b, k, n, d = feats.shape
    ch = 128
    nch = n // ch
    nl = len(ws)
    cnt = float(b * n * k)
    c_in = d // 3

    def body(f_ref, x_ref, *args):
        wfa, wda = args[0], args[1]
        if nl == 2:
            wfb, wdb, s_ref, r_ref, o_ref, st = args[2], args[3], args[4], args[5], args[6], args[7]
        else:
            s_ref, r_ref, o_ref, st = args[2], args[3], args[4], args[5]
        p = pl.program_id(0)
        bi = pl.program_id(1)
        ci = pl.program_id(2)
        first = jnp.logical_and(bi == 0, ci == 0)
        s = s_ref[...]
        r = r_ref[...]

        def build_f():
            xc = x_ref[0]
            fe = f_ref[0]
            fd = (fe - xc[None]).reshape(k * ch, d)
            xr = jnp.broadcast_to(xc[None], (k, ch, d)).reshape(k * ch, d)
            parts = []
            for c in range(3):
                parts.append(fd[:, c * c_in:(c + 1) * c_in])
                parts.append(xr[:, c * c_in:(c + 1) * c_in])
            return jnp.concatenate(parts, axis=1)

        @pl.when(p == 0)
        def _():
            @pl.when(first)
            def _():
                st[...] = jnp.zeros_like(st)
            f = build_f()
            pa = mma(f, wfa[...])
            na = jnp.sqrt(_mm(pa * pa, s)) + EPS
            _acc(st, 0, na)

        @pl.when(p == 1)
        def _():
            f = build_f()
            pa = mma(f, wfa[...])
            if nl == 2:
                @pl.when(first)
                def _():
                    st[pl.ds(2, 2), :] = jnp.zeros((2, st.shape[1]), jnp.float32)
                oa = _vnl(pa, mma(f, wda[...]), st[0:1, :], st[1:2, :], cnt, s, r)
                pb = _mmd(oa, wfb[...])
                nb_ = jnp.sqrt(_mm(pb * pb, s)) + EPS
                _acc(st, 2, nb_)
            else:
                oa = _vnl(pa, mma(f, wda[...]), st[0:1, :], st[1:2, :], cnt, s, r)
                o_ref[0, 0] = jnp.mean(oa.reshape(k, ch, 63), axis=0)

        if nl == 2:
            @pl.when(p == 2)
            def _():
                f = build_f()
                pa = mma(f, wfa[...])
                oa = _vnl(pa, mma(f, wda[...]), st[0:1, :], st[1:2, :], cnt, s, r)
                pb = _mmd(oa, wfb[...])
                ob = _vnl(pb, _mmd(oa, wdb[...]), st[2:3, :], st[3:4, :], cnt, s, r)
                o_ref[0, 0] = jnp.mean(ob.reshape(k, ch, 63), axis=0)

    win = []
    args = []
    for wf, wd in ws:
        win += [_full_spec(wf), _full_spec(wd)]
        args += [wf, wd]
    win += [_full_spec(s21), _full_spec(r21)]
    args += [s21, r21]
    out = pl.pallas_call(
        body,
        grid=(nl + 1, b, nch),
        in_specs=[pl.BlockSpec((1, k, ch, d), lambda p, i, c: (i, 0, c, 0)),
                  pl.BlockSpec((1, ch, d), lambda p, i, c: (i, c, 0))] + win,
        out_specs=pl.BlockSpec((1, 1, ch, 63), lambda p, i, c: (p, i, c, 0)),
        out_shape=jax.ShapeDtypeStruct((nl + 1, b, n, 63), jnp.float32),
        scratch_shapes=[pltpu.VMEM((4, 21), jnp.float32)],
    )(feats, nodes, *args)
    return out[nl]


def _stage_c(x1, x2, x3, w6f3, w6d3, t341, s341, r341):
    b, n, _ = x1.shape
    cnt = float(b * n)

    def body(x1r, x2r, x3r, wf, wd, t_r, s_r, r_r, h_ref, hm_ref, st):
        p = pl.program_id(0)
        bi = pl.program_id(1)
        parts = []
        for c in range(3):
            for xr in (x1r, x2r, x3r):
                parts.append(xr[0][:, c * 21:(c + 1) * 21])
        x123 = jnp.concatenate(parts, axis=1)
        pp = _mmd(x123, wf[...])

        @pl.when(p == 0)
        def _():
            @pl.when(bi == 0)
            def _():
                st[...] = jnp.zeros_like(st)
            na = jnp.sqrt(_mm(pp * pp, s_r[...])) + EPS
            _acc(st, 0, na)

        @pl.when(p == 1)
        def _():
            dd = _mm(x123, wd[...])
            dbig = _mm(dd, t_r[...])
            s = s_r[...]
            r = r_r[...]
            m = st[0:1, :341] / cnt
            var = st[1:2, :341] / cnt - m * m
            na = jnp.sqrt(_mm(pp * pp, s)) + EPS
            nb = (na - m) / jnp.sqrt(var + 1e-5)
            pb = pp * _mm(nb / na, r)
            dot = _mm(pb * dbig, s)
            dns = jnp.sum(dd * dd, axis=1, keepdims=True)
            coef = jnp.where(dot < 0, dot / (dns + EPS), 0.0) * (1.0 - SLOPE)
            h6 = pb - _mm(coef, r) * dbig
            h_ref[0, 0] = h6
            hm_ref[0, 0, 0] = jnp.mean(h6, axis=0)

    h_all, hm_all = pl.pallas_call(
        body,
        grid=(2, b),
        in_specs=[pl.BlockSpec((1, n, 63), lambda p, i: (i, 0, 0))] * 3 +
                 [_full_spec(w6f3), _full_spec(w6d3), _full_spec(t341),
                  _full_spec(s341), _full_spec(r341)],
        out_specs=[pl.BlockSpec((1, 1, n, 1023), lambda p, i: (p, i, 0, 0)),
                   pl.BlockSpec((1, 1, 1, 1023), lambda p, i: (p, i, 0, 0))],
        out_shape=[jax.ShapeDtypeStruct((2, b, n, 1023), jnp.float32),
                   jax.ShapeDtypeStruct((2, b, 1, 1023), jnp.float32)],
        scratch_shapes=[pltpu.VMEM((2, 341), jnp.float32)],
    )(x1, x2, x3, w6f3, w6d3, t341, s341, r341)
    return h_all[1], hm_all[1]


def _hrows(h6_ref, hm_ref, n):
    parts = []
    for c in range(3):
        parts.append(h6_ref[0][:, c * 341:(c + 1) * 341])
        parts.append(jnp.broadcast_to(hm_ref[0, 0:1, c * 341:(c + 1) * 341], (n, 341)))
    return jnp.concatenate(parts, axis=1)


def _stage_d1(h6, hmean, wf3, wd3, s341):
    b, n, _ = h6.shape
    ch = 256
    nch = n // ch

    def body(h_ref, hm_ref, wf, wd, s_r, p_ref, d_ref, st_ref):
        bi = pl.program_id(0)
        ci = pl.program_id(1)
        h = _hrows(h_ref, hm_ref, ch)
        p1 = _mmd(h, wf[...])
        d1 = _mmd(h, wd[...])
        p_ref[0] = p1
        d_ref[0] = d1
        na = jnp.sqrt(_mm(p1 * p1, s_r[...])) + EPS
        s = jnp.sum(na, axis=0, keepdims=True)
        q = jnp.sum(na * na, axis=0, keepdims=True)
        v = jnp.concatenate([s, q], axis=0)[None]
        first = jnp.logical_and(bi == 0, ci == 0)

        @pl.when(first)
        def _():
            st_ref[...] = v

        @pl.when(jnp.logical_not(first))
        def _():
            st_ref[...] += v

    return pl.pallas_call(
        body,
        grid=(b, nch),
        in_specs=[pl.BlockSpec((1, ch, 1023), lambda i, c: (i, c, 0)),
                  pl.BlockSpec((1, 1, 1023), lambda i, c: (i, 0, 0)),
                  _full_spec(wf3), _full_spec(wd3), _full_spec(s341)],
        out_specs=[pl.BlockSpec((1, ch, 1023), lambda i, c: (i, c, 0)),
                   pl.BlockSpec((1, ch, 1023), lambda i, c: (i, c, 0)),
                   pl.BlockSpec((1, 2, 341), lambda i, c: (0, 0, 0))],
        out_shape=[jax.ShapeDtypeStruct((b, n, 1023), jnp.float32),
                   jax.ShapeDtypeStruct((b, n, 1023), jnp.float32),
                   jax.ShapeDtypeStruct((1, 2, 341), jnp.float32)],
    )(h6, hmean, wf3, wd3, s341)


def _stage_d2(p1, d1, st1, wf3, wd3, wlin3, s341, r341, s170, r170):
    b, n, _ = p1.shape
    cnt = float(b * n)

    def body(p_ref, d_ref, st1_ref, wf, wd, wl, s3, r3, s1r, r1r, z_ref, st):
        p = pl.program_id(0)
        bi = pl.program_id(1)
        o1 = _vnl(p_ref[0], d_ref[0], st1_ref[0, 0:1], st1_ref[0, 1:2], cnt,
                  s3[...], r3[...])
        p2 = _mmd(o1, wf[...])

        @pl.when(p == 0)
        def _():
            @pl.when(bi == 0)
            def _():
                st[...] = jnp.zeros_like(st)
            n2 = jnp.sqrt(_mm(p2 * p2, s1r[...])) + EPS
            _acc(st, 0, n2)

        @pl.when(p == 1)
        def _():
            d2 = _mmd(o1, wd[...])
            o2 = _vnl(p2, d2, st[0:1, :], st[1:2, :], cnt, s1r[...], r1r[...])
            z_ref[0, 0] = _mmd(o2, wl[...])

    out = pl.pallas_call(
        body,
        grid=(2, b),
        in_specs=[pl.BlockSpec((1, n, 1023), lambda p, i: (i, 0, 0)),
                  pl.BlockSpec((1, n, 1023), lambda p, i: (i, 0, 0)),
                  pl.BlockSpec((1, 2, 341), lambda p, i: (0, 0, 0)),
                  _full_spec(wf3), _full_spec(wd3), _full_spec(wlin3),
                  _full_spec(s341), _full_spec(r341),
                  _full_spec(s170), _full_spec(r170)],
        out_specs=pl.BlockSpec((1, 1, n, 9), lambda p, i: (p, i, 0, 0)),
        out_shape=jax.ShapeDtypeStruct((2, b, n, 9), jnp.float32),
        scratch_shapes=[pltpu.VMEM((2, 170), jnp.float32)],
    )(p1, d1, st1, wf3, wd3, wlin3, s341, r341, s170, r170)
    return out[1]


def _stage_e(h6, hmean, z0, x1, x2, x3, w8t):
    b, n, _ = h6.shape

    def body(h_ref, hm_ref, z_ref, x1r, x2r, x3r, wt, hx_ref, y_ref):
        z = z_ref[0]
        hs_parts = []
        xe_parts = []
        xp = []
        for c in range(3):
            for xr in (x1r, x2r, x3r):
                xp.append(xr[0][:, c * 21:(c + 1) * 21])
        x123 = jnp.concatenate(xp, axis=1)
        for kk in range(3):
            hs_k = jnp.zeros((n, 682), jnp.float32)
            xe_k = jnp.zeros((n, 63), jnp.float32)
            for c in range(3):
                zc = z[:, c * 3 + kk:c * 3 + kk + 1]
                hcat = jnp.concatenate(
                    [h_ref[0][:, c * 341:(c + 1) * 341],
                     jnp.broadcast_to(hm_ref[0, 0:1, c * 341:(c + 1) * 341], (n, 341))],
                    axis=1)
                hs_k = hs_k + hcat * zc
                xe_k = xe_k + x123[:, c * 63:(c + 1) * 63] * zc
            hs_parts.append(hs_k)
            xe_parts.append(xe_k)
        hs = jnp.concatenate(hs_parts, axis=1)
        xe = jnp.concatenate(xe_parts, axis=1)
        hx_ref[0, 0] = jnp.max(hs, axis=0)
        y_ref[0] = _mmd(xe, wt[...])

    return pl.pallas_call(
        body,
        grid=(b,),
        in_specs=[pl.BlockSpec((1, n, 1023), lambda i: (i, 0, 0)),
                  pl.BlockSpec((1, 1, 1023), lambda i: (i, 0, 0)),
                  pl.BlockSpec((1, n, 9), lambda i: (i, 0, 0)),
                  pl.BlockSpec((1, n, 63), lambda i: (i, 0, 0)),
                  pl.BlockSpec((1, n, 63), lambda i: (i, 0, 0)),
                  pl.BlockSpec((1, n, 63), lambda i: (i, 0, 0)),
                  _full_spec(w8t)],
        out_specs=[pl.BlockSpec((1, 1, 2046), lambda i: (i, 0, 0)),
                   pl.BlockSpec((1, n, 256), lambda i: (i, 0, 0))],
        out_shape=[jax.ShapeDtypeStruct((b, 1, 2046), jnp.float32),
                   jax.ShapeDtypeStruct((b, n, 256), jnp.float32)],
    )(h6, hmean, z0, x1, x2, x3, w8t)


def _stage_f(y8p, hmax, lr, w7, w8h, w8m, w9, w10, w11):
    b, n, _ = y8p.shape
    cnt = float(b * n)

    def cb(y, st, row, width):
        m = st[pl.ds(row, 1), :width] / cnt
        var = st[pl.ds(row + 1, 1), :width] / cnt - m * m
        h = (y - m) / jnp.sqrt(var + 1e-5)
        return jnp.where(h >= 0, h, SLOPE * h)

    def body(y_ref, hx_ref, l_ref, w7r, w8hr, w8mr, w9r, w10r, w11r, o_ref, st):
        p = pl.program_id(0)
        bi = pl.program_id(1)
        lfull = _mmd(l_ref[0], w7r[...])
        m7 = jnp.mean(lfull, axis=0, keepdims=True)
        v7 = jnp.mean(lfull * lfull, axis=0, keepdims=True) - m7 * m7
        lf = (lfull - m7) / jnp.sqrt(v7 + 1e-5)
        lf = jnp.where(lf >= 0, lf, SLOPE * lf)
        sel = lax.broadcasted_iota(jnp.int32, (8, 1), 0) == bi
        lfb = jnp.sum(jnp.where(sel, lf, 0.0), axis=0, keepdims=True)
        bvec = _mmd(hx_ref[0], w8hr[...]) + _mmd(lfb, w8mr[...])
        y8 = y_ref[0] + bvec

        @pl.when(p == 0)
        def _():
            @pl.when(bi == 0)
            def _():
                st[...] = jnp.zeros_like(st)
            _acc(st, 0, y8)

        @pl.when(p == 1)
        def _():
            y9 = _mmd(cb(y8, st, 0, 256), w9r[...])
            _acc(st, 2, y9)

        @pl.when(p == 2)
        def _():
            y9 = _mmd(cb(y8, st, 0, 256), w9r[...])
            y10 = _mmd(cb(y9, st, 2, 256), w10r[...])
            st[pl.ds(4, 1), :128] += jnp.sum(y10, axis=0, keepdims=True)
            st[pl.ds(5, 1), :128] += jnp.sum(y10 * y10, axis=0, keepdims=True)

        @pl.when(p == 3)
        def _():
            y9 = _mmd(cb(y8, st, 0, 256), w9r[...])
            y10 = _mmd(cb(y9, st, 2, 256), w10r[...])
            y11 = _mmd(cb(y10, st, 4, 128), w11r[...])
            st[pl.ds(6, 1), :50] += jnp.sum(y11, axis=0, keepdims=True)
            st[pl.ds(7, 1), :50] += jnp.sum(y11 * y11, axis=0, keepdims=True)

        @pl.when(p == 4)
        def _():
            y9 = _mmd(cb(y8, st, 0, 256), w9r[...])
            y10 = _mmd(cb(y9, st, 2, 256), w10r[...])
            y11 = _mmd(cb(y10, st, 4, 128), w11r[...])
            m = st[6:7, :50] / cnt
            var = st[7:8, :50] / cnt - m * m
            o_ref[0, 0] = (y11 - m) / jnp.sqrt(var + 1e-5)

    out = pl.pallas_call(
        body,
        grid=(5, b),
        in_specs=[pl.BlockSpec((1, n, 256), lambda p, i: (i, 0, 0)),
                  pl.BlockSpec((1, 1, 2046), lambda p, i: (i, 0, 0)),
                  pl.BlockSpec((1, 8, 16), lambda p, i: (0, 0, 0)),
                  _full_spec(w7), _full_spec(w8h), _full_spec(w8m),
                  _full_spec(w9), _full_spec(w10), _full_spec(w11)],
        out_specs=pl.BlockSpec((1, 1, n, 50), lambda p, i: (p, i, 0, 0)),
        out_shape=jax.ShapeDtypeStruct((5, b, n, 50), jnp.float32),
        scratch_shapes=[pltpu.VMEM((8, 256), jnp.float32)],
    )(y8p, hmax, lr, w7, w8h, w8m, w9, w10, w11)
    return out[4]


def kernel(x, l, params):
    P = params
    b, _, n = x.shape
    s21, r21 = _sel(21)
    s341, r341 = _sel(341)
    s170, r170 = _sel(170)
    t341 = jnp.kron(jnp.eye(3, dtype=jnp.float32), jnp.ones((1, 341), jnp.float32))

    nodes0 = x.transpose(0, 2, 1)
    idx1 = _knn(nodes0)
    feat1 = _gather(nodes0, idx1)
    x1 = _edge_block(feat1, nodes0,
                     [(_kron3(P['w1f']), _kron3(P['w1d'])),
                      (_kron3(P['w2f']), _kron3(P['w2d']))], s21, r21,
                     hi_a=True)
    idx2 = _knn(x1)
    feat2 = _gather(x1, idx2)
    x2 = _edge_block(feat2, x1,
                     [(_kron3(P['w3f']), _kron3(P['w3d'])),
                      (_kron3(P['w4f']), _kron3(P['w4d']))], s21, r21)
    idx3 = _knn(x2)
    feat3 = _gather(x2, idx3)
    x3 = _edge_block(feat3, x2,
                     [(_kron3(P['w5f']), _kron3(P['w5d']))], s21, r21)

    h6, hmean = _stage_c(x1, x2, x3, _kron3(P['w6f']), _kron3(P['w6d']),
                         t341, s341, r341)
    p1, d1, st1 = _stage_d1(h6, hmean, _kron3(P['ws1f']), _kron3(P['ws1d']), s341)
    z0 = _stage_d2(p1, d1, st1, _kron3(P['ws2f']), _kron3(P['ws2d']),
                   _kron3(P['wlin']), s341, r341, s170, r170)

    w8 = P['w8']
    w8h = w8[:2046].reshape(682, 3, 256).transpose(1, 0, 2).reshape(2046, 256)
    w8m = w8[2046:2110]
    w8t = w8[2110:2299].reshape(63, 3, 256).transpose(1, 0, 2).reshape(189, 256)
    hmax, y8p = _stage_e(h6, hmean, z0, x1, x2, x3, w8t)
    lr = l.reshape(1, b, 16)
    return _stage_f(y8p, hmax, lr, P['w7'], w8h, w8m, P['w9'], P['w10'], P['w11'])
